# baseline (speedup 1.0000x reference)
import functools

import jax
import jax.numpy as jnp
from jax import lax
from jax.experimental import pallas as pl
from jax.experimental.pallas import tpu as pltpu
from jax.experimental.pallas import tpu_sc as plsc

N = 10000
NP = 10240
E = 320000
IN_DIM = 128
HIDDEN = 256
HH = HIDDEN // 2
NUM_CLASSES = 2
NUM_GRAPHS = 64

NC = 2
NS = 16
NW = NC * NS
EC = 80
ECA = 80
NBUF = 2
SLAB = NP // NS

_MESH = plsc.VectorSubcoreMesh(core_axis_name="c", subcore_axis_name="s")

ROW_BLK = 1024
GRID_R = NP // ROW_BLK


def _zero_vmem_1d(ref, n):
    def body(i, carry):
        ref[pl.ds(i * 16, 16)] = jnp.zeros((16,), jnp.float32)
        return carry
    lax.fori_loop(0, n // 16, body, 0)


@functools.partial(
    pl.kernel,
    out_type=(jax.ShapeDtypeStruct((NP,), jnp.float32),
              jax.ShapeDtypeStruct((NP,), jnp.float32)),
    mesh=_MESH,
    scratch_types=(
        pltpu.VMEM((E // NW,), jnp.int32),
        pltpu.VMEM((EC,), jnp.float32),
        pltpu.VMEM((SLAB,), jnp.float32),
        pltpu.VMEM_SHARED((NP,), jnp.float32),
        pltpu.SemaphoreType.DMA,
    ),
)
def _deg_kernel(dst_hbm, deg0_hbm, deg1_hbm, idx_v, ones_v, zeros_v, deg_sh, sem):
    c = lax.axis_index("c")
    s = lax.axis_index("s")
    for j in range(EC // 16):
        ones_v[pl.ds(j * 16, 16)] = jnp.ones((16,), jnp.float32)
    _zero_vmem_1d(zeros_v, SLAB)
    pltpu.sync_copy(zeros_v, deg_sh.at[pl.ds(s * SLAB, SLAB)])
    per_w = E // NW
    pltpu.sync_copy(dst_hbm.at[pl.ds((c * NS + s) * per_w, per_w)], idx_v)
    plsc.subcore_barrier()

    def fire(i, carry):
        pltpu.async_copy(ones_v, deg_sh.at[idx_v.at[pl.ds(i * EC, EC)]],
                         sem, add=True)
        return carry
    lax.fori_loop(0, per_w // EC, fire, 0)

    def drain(i, carry):
        pltpu.make_async_copy(ones_v,
                              deg_sh.at[idx_v.at[pl.ds(i * EC, EC)]],
                              sem).wait()
        return carry
    lax.fori_loop(0, per_w // EC, drain, 0)

    plsc.subcore_barrier()

    @pl.when(c == 0)
    def _():
        pltpu.sync_copy(deg_sh.at[pl.ds(s * SLAB, SLAB)],
                        deg0_hbm.at[pl.ds(s * SLAB, SLAB)])

    @pl.when(c == 1)
    def _():
        pltpu.sync_copy(deg_sh.at[pl.ds(s * SLAB, SLAB)],
                        deg1_hbm.at[pl.ds(s * SLAB, SLAB)])


def _agg_pipelined(y_hbm, src_hbm, dst_hbm, z_sh, idxs_v, idxd_v,
                   rows, semg, semc, ebase, n_e):
    pltpu.sync_copy(src_hbm.at[pl.ds(ebase, n_e)], idxs_v)
    pltpu.sync_copy(dst_hbm.at[pl.ds(ebase, n_e)], idxd_v)
    nch = n_e // ECA

    def s_idx(i):
        return idxs_v.at[pl.ds(i * ECA, ECA)]

    def d_idx(i):
        return idxd_v.at[pl.ds(i * ECA, ECA)]

    for b in range(NBUF):
        pltpu.async_copy(y_hbm.at[s_idx(b)], rows[b], semg[b])

    def step(i, b, bp):
        pltpu.make_async_copy(y_hbm.at[s_idx(i)], rows[b],
                              semg[b]).wait()
        pltpu.async_copy(rows[b], z_sh.at[d_idx(i)], semc[b],
                         add=True)
        ip = jnp.maximum(i - 1, 0)

        @pl.when(i >= 1)
        def _():
            pltpu.make_async_copy(rows[bp], z_sh.at[d_idx(ip)],
                                  semc[bp]).wait()

        @pl.when((i >= 1) & (i - 1 + NBUF < nch))
        def _():
            pltpu.async_copy(y_hbm.at[s_idx(ip + NBUF)], rows[bp],
                             semg[bp])

    def body(ib, carry):
        for b in range(NBUF):
            step(ib * NBUF + b, b, (b - 1) % NBUF)
        return carry
    lax.fori_loop(0, nch // NBUF, body, 0)
    for r in range(nch % NBUF):
        i = nch - (nch % NBUF) + r
        step(i, i % NBUF, (i - 1) % NBUF)
    pltpu.make_async_copy(rows[(nch - 1) % NBUF], z_sh.at[d_idx(nch - 1)],
                          semc[(nch - 1) % NBUF]).wait()


def _agg_prologue(z_sh, rows, s):
    def body(i, carry):
        rows[i // (IN_DIM // 16), pl.ds((i % (IN_DIM // 16)) * 16, 16)] = (
            jnp.zeros((16,), jnp.float32))
        return carry
    lax.fori_loop(0, ECA * IN_DIM // 16, body, 0)
    for j in range(SLAB // ECA):
        pltpu.sync_copy(rows, z_sh.at[pl.ds(s * SLAB + j * ECA, ECA)])
    plsc.subcore_barrier()


EPT = E // NW


def _agg_scratch(n_e_per):
    return (
        pltpu.VMEM((n_e_per,), jnp.int32),
        pltpu.VMEM((n_e_per,), jnp.int32),
        *[pltpu.VMEM((ECA, IN_DIM), jnp.float32) for _ in range(NBUF)],
        pltpu.VMEM_SHARED((NP, IN_DIM), jnp.float32),
        *[pltpu.SemaphoreType.DMA for _ in range(2 * NBUF)],
    )


_AGG_OUT = (jax.ShapeDtypeStruct((NP, IN_DIM), jnp.float32),
            jax.ShapeDtypeStruct((NP, IN_DIM), jnp.float32))


@functools.partial(pl.kernel, out_type=_AGG_OUT, mesh=_MESH,
                   scratch_types=_agg_scratch(EPT))
def _agg1_kernel(y_hbm, src_hbm, dst_hbm, z0_hbm, z1_hbm,
                 idxs_v, idxd_v, *rs):
    rows, z_sh = rs[:NBUF], rs[NBUF]
    semg, semc = rs[NBUF + 1:NBUF + 1 + NBUF], rs[NBUF + 1 + NBUF:]
    c = lax.axis_index("c")
    s = lax.axis_index("s")
    _agg_prologue(z_sh, rows[0], s)
    ebase = (c * NS + s) * EPT
    _agg_pipelined(y_hbm, src_hbm, dst_hbm, z_sh, idxs_v, idxd_v,
                   rows, semg, semc, ebase, EPT)
    plsc.subcore_barrier()

    @pl.when(c == 0)
    def _():
        pltpu.sync_copy(z_sh.at[pl.ds(s * SLAB, SLAB)],
                        z0_hbm.at[pl.ds(s * SLAB, SLAB)])

    @pl.when(c == 1)
    def _():
        pltpu.sync_copy(z_sh.at[pl.ds(s * SLAB, SLAB)],
                        z1_hbm.at[pl.ds(s * SLAB, SLAB)])


@functools.partial(pl.kernel, out_type=_AGG_OUT, mesh=_MESH,
                   scratch_types=_agg_scratch(EPT))
def _agg2_kernel(ya_hbm, yb_hbm, src_hbm, dst_hbm, za_hbm, zb_hbm,
                 idxs_v, idxd_v, *rs):
    rows, z_sh = rs[:NBUF], rs[NBUF]
    semg, semc = rs[NBUF + 1:NBUF + 1 + NBUF], rs[NBUF + 1 + NBUF:]
    c = lax.axis_index("c")
    s = lax.axis_index("s")
    _agg_prologue(z_sh, rows[0], s)
    ebase = s * (E // NS)

    @pl.when(c == 0)
    def _():
        for p in range(E // NS // EPT):
            _agg_pipelined(ya_hbm, src_hbm, dst_hbm, z_sh, idxs_v, idxd_v,
                           rows, semg, semc, ebase + p * EPT, EPT)

    @pl.when(c == 1)
    def _():
        for p in range(E // NS // EPT):
            _agg_pipelined(yb_hbm, src_hbm, dst_hbm, z_sh, idxs_v, idxd_v,
                           rows, semg, semc, ebase + p * EPT, EPT)

    plsc.subcore_barrier()

    @pl.when(c == 0)
    def _():
        pltpu.sync_copy(z_sh.at[pl.ds(s * SLAB, SLAB)],
                        za_hbm.at[pl.ds(s * SLAB, SLAB)])

    @pl.when(c == 1)
    def _():
        pltpu.sync_copy(z_sh.at[pl.ds(s * SLAB, SLAB)],
                        zb_hbm.at[pl.ds(s * SLAB, SLAB)])


def _scale_body(deg0_ref, deg1_ref, x_ref, dinv_ref, y1_ref):
    deg = deg0_ref[...] + deg1_ref[...] + 1.0
    dinv = lax.rsqrt(deg)
    dinv_ref[...] = dinv
    y1_ref[...] = x_ref[...] * dinv


_scale_call = pl.pallas_call(
    _scale_body,
    grid=(GRID_R,),
    in_specs=[
        pl.BlockSpec((ROW_BLK, 1), lambda i: (i, 0)),
        pl.BlockSpec((ROW_BLK, 1), lambda i: (i, 0)),
        pl.BlockSpec((ROW_BLK, IN_DIM), lambda i: (i, 0)),
    ],
    out_specs=[
        pl.BlockSpec((ROW_BLK, 1), lambda i: (i, 0)),
        pl.BlockSpec((ROW_BLK, IN_DIM), lambda i: (i, 0)),
    ],
    out_shape=[
        jax.ShapeDtypeStruct((NP, 1), jnp.float32),
        jax.ShapeDtypeStruct((NP, IN_DIM), jnp.float32),
    ],
)


def _layer1_body(z0_ref, z1_ref, y1_ref, dinv_ref, w1_ref, b1_ref,
                 ya_ref, yb_ref):
    dinv = dinv_ref[...]
    t = (z0_ref[...] + z1_ref[...] + y1_ref[...]) * dinv
    h = jnp.dot(t, w1_ref[...], preferred_element_type=jnp.float32)
    h = jnp.maximum(h + b1_ref[...], 0.0)
    y2 = h * dinv
    ya_ref[...] = y2[:, :HH]
    yb_ref[...] = y2[:, HH:]


_layer1_call = pl.pallas_call(
    _layer1_body,
    grid=(GRID_R,),
    in_specs=[
        pl.BlockSpec((ROW_BLK, IN_DIM), lambda i: (i, 0)),
        pl.BlockSpec((ROW_BLK, IN_DIM), lambda i: (i, 0)),
        pl.BlockSpec((ROW_BLK, IN_DIM), lambda i: (i, 0)),
        pl.BlockSpec((ROW_BLK, 1), lambda i: (i, 0)),
        pl.BlockSpec((IN_DIM, HIDDEN), lambda i: (0, 0)),
        pl.BlockSpec((1, HIDDEN), lambda i: (0, 0)),
    ],
    out_specs=[
        pl.BlockSpec((ROW_BLK, HH), lambda i: (i, 0)),
        pl.BlockSpec((ROW_BLK, HH), lambda i: (i, 0)),
    ],
    out_shape=[
        jax.ShapeDtypeStruct((NP, HH), jnp.float32),
        jax.ShapeDtypeStruct((NP, HH), jnp.float32),
    ],
)


def _readout_body(za_ref, zb_ref, ya_ref, yb_ref, dinv_ref, batch_ref,
                  w2_ref, b2_ref, lw_ref, lb_ref, out_ref, gsum, gmax):
    i = pl.program_id(0)

    @pl.when(i == 0)
    def _():
        gsum[...] = jnp.zeros_like(gsum)
        gmax[...] = jnp.zeros_like(gmax)

    dinv = dinv_ref[...]
    t = jnp.concatenate(
        [za_ref[...] + ya_ref[...], zb_ref[...] + yb_ref[...]], axis=1)
    t = t * dinv
    h = jnp.dot(t, w2_ref[...], preferred_element_type=jnp.float32)
    h = jnp.maximum(h + b2_ref[...], 0.0)

    b = batch_ref[...]
    onehot = (b == lax.broadcasted_iota(jnp.int32, (1, NUM_GRAPHS), 1)
              ).astype(jnp.float32)
    gsum[...] += lax.dot_general(onehot, h, (((0,), (0,)), ((), ())),
                                 preferred_element_type=jnp.float32)

    def seg_max(g, carry):
        v = jnp.max(jnp.where(b == g, h, 0.0), axis=0, keepdims=True)
        cur = gmax[pl.ds(g, 1), :]
        gmax[pl.ds(g, 1), :] = jnp.maximum(cur, v)
        return carry
    lo = batch_ref[0, 0]
    hi = jnp.minimum(batch_ref[ROW_BLK - 1, 0], NUM_GRAPHS - 1) + 1
    lax.fori_loop(lo, hi, seg_max, 0)

    @pl.when(i == GRID_R - 1)
    def _():
        g = jnp.concatenate([gsum[...], gmax[...]], axis=1)
        out_ref[...] = (jnp.dot(g, lw_ref[...],
                                preferred_element_type=jnp.float32)
                        + lb_ref[...])


_readout_call = pl.pallas_call(
    _readout_body,
    grid=(GRID_R,),
    in_specs=[
        pl.BlockSpec((ROW_BLK, HH), lambda i: (i, 0)),
        pl.BlockSpec((ROW_BLK, HH), lambda i: (i, 0)),
        pl.BlockSpec((ROW_BLK, HH), lambda i: (i, 0)),
        pl.BlockSpec((ROW_BLK, HH), lambda i: (i, 0)),
        pl.BlockSpec((ROW_BLK, 1), lambda i: (i, 0)),
        pl.BlockSpec((ROW_BLK, 1), lambda i: (i, 0)),
        pl.BlockSpec((HIDDEN, HIDDEN), lambda i: (0, 0)),
        pl.BlockSpec((1, HIDDEN), lambda i: (0, 0)),
        pl.BlockSpec((2 * HIDDEN, NUM_CLASSES), lambda i: (0, 0)),
        pl.BlockSpec((1, NUM_CLASSES), lambda i: (0, 0)),
    ],
    out_specs=pl.BlockSpec((NUM_GRAPHS, NUM_CLASSES), lambda i: (0, 0)),
    out_shape=jax.ShapeDtypeStruct((NUM_GRAPHS, NUM_CLASSES), jnp.float32),
    scratch_shapes=[
        pltpu.VMEM((NUM_GRAPHS, HIDDEN), jnp.float32),
        pltpu.VMEM((NUM_GRAPHS, HIDDEN), jnp.float32),
    ],
)


def kernel(x, edge_index, batch, W1, b1, W2, b2, lin_W, lin_b):
    src = edge_index[0]
    dst = edge_index[1]
    x_p = jnp.pad(x, ((0, NP - N), (0, 0)))
    batch_p = jnp.pad(batch, (0, NP - N), constant_values=NUM_GRAPHS)

    deg0, deg1 = _deg_kernel(dst)
    dinv, y1 = _scale_call(deg0.reshape(NP, 1), deg1.reshape(NP, 1), x_p)
    z1p0, z1p1 = _agg1_kernel(y1, src, dst)
    y2a, y2b = _layer1_call(z1p0, z1p1, y1, dinv, W1, b1.reshape(1, HIDDEN))
    z2a, z2b = _agg2_kernel(y2a, y2b, src, dst)
    return _readout_call(z2a, z2b, y2a, y2b, dinv, batch_p.reshape(NP, 1),
                         W2, b2.reshape(1, HIDDEN), lin_W,
                         lin_b.reshape(1, NUM_CLASSES))


# speedup vs baseline: 1.4680x; 1.4680x over previous
import functools

import jax
import jax.numpy as jnp
from jax import lax
from jax.experimental import pallas as pl
from jax.experimental.pallas import tpu as pltpu
from jax.experimental.pallas import tpu_sc as plsc

N = 10000
NP = 10240
E = 320000
IN_DIM = 128
HIDDEN = 256
HH = HIDDEN // 2
NUM_CLASSES = 2
NUM_GRAPHS = 64

NC = 2
NS = 16
NW = NC * NS
EC = 80
ECA = 40
NBUF = 5
SLAB = NP // NS

_MESH = plsc.VectorSubcoreMesh(core_axis_name="c", subcore_axis_name="s")

ROW_BLK = 1024
GRID_R = NP // ROW_BLK


def _zero_vmem_1d(ref, n):
    def body(i, carry):
        ref[pl.ds(i * 16, 16)] = jnp.zeros((16,), jnp.float32)
        return carry
    lax.fori_loop(0, n // 16, body, 0)


@functools.partial(
    pl.kernel,
    out_type=(jax.ShapeDtypeStruct((NP,), jnp.float32),
              jax.ShapeDtypeStruct((NP,), jnp.float32)),
    mesh=_MESH,
    scratch_types=(
        pltpu.VMEM((E // NW,), jnp.int32),
        pltpu.VMEM((EC,), jnp.float32),
        pltpu.VMEM((SLAB,), jnp.float32),
        pltpu.VMEM_SHARED((NP,), jnp.float32),
        pltpu.SemaphoreType.DMA,
    ),
)
def _deg_kernel(dst_hbm, deg0_hbm, deg1_hbm, idx_v, ones_v, zeros_v, deg_sh, sem):
    c = lax.axis_index("c")
    s = lax.axis_index("s")
    for j in range(EC // 16):
        ones_v[pl.ds(j * 16, 16)] = jnp.ones((16,), jnp.float32)
    _zero_vmem_1d(zeros_v, SLAB)
    pltpu.sync_copy(zeros_v, deg_sh.at[pl.ds(s * SLAB, SLAB)])
    per_w = E // NW
    pltpu.sync_copy(dst_hbm.at[pl.ds((c * NS + s) * per_w, per_w)], idx_v)
    plsc.subcore_barrier()

    def fire(i, carry):
        pltpu.async_copy(ones_v, deg_sh.at[idx_v.at[pl.ds(i * EC, EC)]],
                         sem, add=True)
        return carry
    lax.fori_loop(0, per_w // EC, fire, 0)

    def drain(i, carry):
        pltpu.make_async_copy(ones_v,
                              deg_sh.at[idx_v.at[pl.ds(i * EC, EC)]],
                              sem).wait()
        return carry
    lax.fori_loop(0, per_w // EC, drain, 0)

    plsc.subcore_barrier()

    @pl.when(c == 0)
    def _():
        pltpu.sync_copy(deg_sh.at[pl.ds(s * SLAB, SLAB)],
                        deg0_hbm.at[pl.ds(s * SLAB, SLAB)])

    @pl.when(c == 1)
    def _():
        pltpu.sync_copy(deg_sh.at[pl.ds(s * SLAB, SLAB)],
                        deg1_hbm.at[pl.ds(s * SLAB, SLAB)])


def _agg_pipelined(y_hbm, src_hbm, dst_hbm, z_sh, idxs_v, idxd_v,
                   rows, semg, semc, ebase, n_e):
    pltpu.sync_copy(src_hbm.at[pl.ds(ebase, n_e)], idxs_v)
    pltpu.sync_copy(dst_hbm.at[pl.ds(ebase, n_e)], idxd_v)
    nch = n_e // ECA

    def s_idx(i):
        return idxs_v.at[pl.ds(i * ECA, ECA)]

    def d_idx(i):
        return idxd_v.at[pl.ds(i * ECA, ECA)]

    for b in range(NBUF):
        pltpu.async_copy(y_hbm.at[s_idx(b)], rows[b], semg[b])

    def step(i, b, bp):
        pltpu.make_async_copy(y_hbm.at[s_idx(i)], rows[b],
                              semg[b]).wait()
        pltpu.async_copy(rows[b], z_sh.at[d_idx(i)], semc[b],
                         add=True)
        ip = jnp.maximum(i - 1, 0)

        @pl.when(i >= 1)
        def _():
            pltpu.make_async_copy(rows[bp], z_sh.at[d_idx(ip)],
                                  semc[bp]).wait()

        @pl.when((i >= 1) & (i - 1 + NBUF < nch))
        def _():
            pltpu.async_copy(y_hbm.at[s_idx(ip + NBUF)], rows[bp],
                             semg[bp])

    def body(ib, carry):
        for b in range(NBUF):
            step(ib * NBUF + b, b, (b - 1) % NBUF)
        return carry
    lax.fori_loop(0, nch // NBUF, body, 0)
    for r in range(nch % NBUF):
        i = nch - (nch % NBUF) + r
        step(i, i % NBUF, (i - 1) % NBUF)
    pltpu.make_async_copy(rows[(nch - 1) % NBUF], z_sh.at[d_idx(nch - 1)],
                          semc[(nch - 1) % NBUF]).wait()


def _agg_prologue(z_sh, rows, s):
    def body(i, carry):
        rows[i // (IN_DIM // 16), pl.ds((i % (IN_DIM // 16)) * 16, 16)] = (
            jnp.zeros((16,), jnp.float32))
        return carry
    lax.fori_loop(0, ECA * IN_DIM // 16, body, 0)
    for j in range(SLAB // ECA):
        pltpu.sync_copy(rows, z_sh.at[pl.ds(s * SLAB + j * ECA, ECA)])
    plsc.subcore_barrier()


EPT = E // NW


def _agg_scratch(n_e_per):
    return (
        pltpu.VMEM((n_e_per,), jnp.int32),
        pltpu.VMEM((n_e_per,), jnp.int32),
        *[pltpu.VMEM((ECA, IN_DIM), jnp.float32) for _ in range(NBUF)],
        pltpu.VMEM_SHARED((NP, IN_DIM), jnp.float32),
        *[pltpu.SemaphoreType.DMA for _ in range(2 * NBUF)],
    )


_AGG_OUT = (jax.ShapeDtypeStruct((NP, IN_DIM), jnp.float32),
            jax.ShapeDtypeStruct((NP, IN_DIM), jnp.float32))


@functools.partial(pl.kernel, out_type=_AGG_OUT, mesh=_MESH,
                   scratch_types=_agg_scratch(EPT))
def _agg1_kernel(y_hbm, src_hbm, dst_hbm, z0_hbm, z1_hbm,
                 idxs_v, idxd_v, *rs):
    rows, z_sh = rs[:NBUF], rs[NBUF]
    semg, semc = rs[NBUF + 1:NBUF + 1 + NBUF], rs[NBUF + 1 + NBUF:]
    c = lax.axis_index("c")
    s = lax.axis_index("s")
    _agg_prologue(z_sh, rows[0], s)
    ebase = (c * NS + s) * EPT
    _agg_pipelined(y_hbm, src_hbm, dst_hbm, z_sh, idxs_v, idxd_v,
                   rows, semg, semc, ebase, EPT)
    plsc.subcore_barrier()

    @pl.when(c == 0)
    def _():
        pltpu.sync_copy(z_sh.at[pl.ds(s * SLAB, SLAB)],
                        z0_hbm.at[pl.ds(s * SLAB, SLAB)])

    @pl.when(c == 1)
    def _():
        pltpu.sync_copy(z_sh.at[pl.ds(s * SLAB, SLAB)],
                        z1_hbm.at[pl.ds(s * SLAB, SLAB)])


@functools.partial(pl.kernel, out_type=_AGG_OUT, mesh=_MESH,
                   scratch_types=_agg_scratch(EPT))
def _agg2_kernel(ya_hbm, yb_hbm, src_hbm, dst_hbm, za_hbm, zb_hbm,
                 idxs_v, idxd_v, *rs):
    rows, z_sh = rs[:NBUF], rs[NBUF]
    semg, semc = rs[NBUF + 1:NBUF + 1 + NBUF], rs[NBUF + 1 + NBUF:]
    c = lax.axis_index("c")
    s = lax.axis_index("s")
    _agg_prologue(z_sh, rows[0], s)
    ebase = s * (E // NS)

    @pl.when(c == 0)
    def _():
        for p in range(E // NS // EPT):
            _agg_pipelined(ya_hbm, src_hbm, dst_hbm, z_sh, idxs_v, idxd_v,
                           rows, semg, semc, ebase + p * EPT, EPT)

    @pl.when(c == 1)
    def _():
        for p in range(E // NS // EPT):
            _agg_pipelined(yb_hbm, src_hbm, dst_hbm, z_sh, idxs_v, idxd_v,
                           rows, semg, semc, ebase + p * EPT, EPT)

    plsc.subcore_barrier()

    @pl.when(c == 0)
    def _():
        pltpu.sync_copy(z_sh.at[pl.ds(s * SLAB, SLAB)],
                        za_hbm.at[pl.ds(s * SLAB, SLAB)])

    @pl.when(c == 1)
    def _():
        pltpu.sync_copy(z_sh.at[pl.ds(s * SLAB, SLAB)],
                        zb_hbm.at[pl.ds(s * SLAB, SLAB)])


def _scale_body(deg0_ref, deg1_ref, x_ref, dinv_ref, y1_ref):
    deg = deg0_ref[...] + deg1_ref[...] + 1.0
    dinv = lax.rsqrt(deg)
    dinv_ref[...] = dinv
    y1_ref[...] = x_ref[...] * dinv


_scale_call = pl.pallas_call(
    _scale_body,
    grid=(GRID_R,),
    in_specs=[
        pl.BlockSpec((ROW_BLK, 1), lambda i: (i, 0)),
        pl.BlockSpec((ROW_BLK, 1), lambda i: (i, 0)),
        pl.BlockSpec((ROW_BLK, IN_DIM), lambda i: (i, 0)),
    ],
    out_specs=[
        pl.BlockSpec((ROW_BLK, 1), lambda i: (i, 0)),
        pl.BlockSpec((ROW_BLK, IN_DIM), lambda i: (i, 0)),
    ],
    out_shape=[
        jax.ShapeDtypeStruct((NP, 1), jnp.float32),
        jax.ShapeDtypeStruct((NP, IN_DIM), jnp.float32),
    ],
)


def _layer1_body(z0_ref, z1_ref, y1_ref, dinv_ref, w1_ref, b1_ref,
                 ya_ref, yb_ref):
    dinv = dinv_ref[...]
    t = (z0_ref[...] + z1_ref[...] + y1_ref[...]) * dinv
    h = jnp.dot(t, w1_ref[...], preferred_element_type=jnp.float32)
    h = jnp.maximum(h + b1_ref[...], 0.0)
    y2 = h * dinv
    ya_ref[...] = y2[:, :HH]
    yb_ref[...] = y2[:, HH:]


_layer1_call = pl.pallas_call(
    _layer1_body,
    grid=(GRID_R,),
    in_specs=[
        pl.BlockSpec((ROW_BLK, IN_DIM), lambda i: (i, 0)),
        pl.BlockSpec((ROW_BLK, IN_DIM), lambda i: (i, 0)),
        pl.BlockSpec((ROW_BLK, IN_DIM), lambda i: (i, 0)),
        pl.BlockSpec((ROW_BLK, 1), lambda i: (i, 0)),
        pl.BlockSpec((IN_DIM, HIDDEN), lambda i: (0, 0)),
        pl.BlockSpec((1, HIDDEN), lambda i: (0, 0)),
    ],
    out_specs=[
        pl.BlockSpec((ROW_BLK, HH), lambda i: (i, 0)),
        pl.BlockSpec((ROW_BLK, HH), lambda i: (i, 0)),
    ],
    out_shape=[
        jax.ShapeDtypeStruct((NP, HH), jnp.float32),
        jax.ShapeDtypeStruct((NP, HH), jnp.float32),
    ],
)


def _readout_body(za_ref, zb_ref, ya_ref, yb_ref, dinv_ref, batch_ref,
                  w2_ref, b2_ref, lw_ref, lb_ref, out_ref, gsum, gmax):
    i = pl.program_id(0)

    @pl.when(i == 0)
    def _():
        gsum[...] = jnp.zeros_like(gsum)
        gmax[...] = jnp.zeros_like(gmax)

    dinv = dinv_ref[...]
    t = jnp.concatenate(
        [za_ref[...] + ya_ref[...], zb_ref[...] + yb_ref[...]], axis=1)
    t = t * dinv
    h = jnp.dot(t, w2_ref[...], preferred_element_type=jnp.float32)
    h = jnp.maximum(h + b2_ref[...], 0.0)

    b = batch_ref[...]
    onehot = (b == lax.broadcasted_iota(jnp.int32, (1, NUM_GRAPHS), 1)
              ).astype(jnp.float32)
    gsum[...] += lax.dot_general(onehot, h, (((0,), (0,)), ((), ())),
                                 preferred_element_type=jnp.float32)

    def seg_max(g, carry):
        v = jnp.max(jnp.where(b == g, h, 0.0), axis=0, keepdims=True)
        cur = gmax[pl.ds(g, 1), :]
        gmax[pl.ds(g, 1), :] = jnp.maximum(cur, v)
        return carry
    lo = batch_ref[0, 0]
    hi = jnp.minimum(batch_ref[ROW_BLK - 1, 0], NUM_GRAPHS - 1) + 1
    lax.fori_loop(lo, hi, seg_max, 0)

    @pl.when(i == GRID_R - 1)
    def _():
        g = jnp.concatenate([gsum[...], gmax[...]], axis=1)
        out_ref[...] = (jnp.dot(g, lw_ref[...],
                                preferred_element_type=jnp.float32)
                        + lb_ref[...])


_readout_call = pl.pallas_call(
    _readout_body,
    grid=(GRID_R,),
    in_specs=[
        pl.BlockSpec((ROW_BLK, HH), lambda i: (i, 0)),
        pl.BlockSpec((ROW_BLK, HH), lambda i: (i, 0)),
        pl.BlockSpec((ROW_BLK, HH), lambda i: (i, 0)),
        pl.BlockSpec((ROW_BLK, HH), lambda i: (i, 0)),
        pl.BlockSpec((ROW_BLK, 1), lambda i: (i, 0)),
        pl.BlockSpec((ROW_BLK, 1), lambda i: (i, 0)),
        pl.BlockSpec((HIDDEN, HIDDEN), lambda i: (0, 0)),
        pl.BlockSpec((1, HIDDEN), lambda i: (0, 0)),
        pl.BlockSpec((2 * HIDDEN, NUM_CLASSES), lambda i: (0, 0)),
        pl.BlockSpec((1, NUM_CLASSES), lambda i: (0, 0)),
    ],
    out_specs=pl.BlockSpec((NUM_GRAPHS, NUM_CLASSES), lambda i: (0, 0)),
    out_shape=jax.ShapeDtypeStruct((NUM_GRAPHS, NUM_CLASSES), jnp.float32),
    scratch_shapes=[
        pltpu.VMEM((NUM_GRAPHS, HIDDEN), jnp.float32),
        pltpu.VMEM((NUM_GRAPHS, HIDDEN), jnp.float32),
    ],
)


def kernel(x, edge_index, batch, W1, b1, W2, b2, lin_W, lin_b):
    src = edge_index[0]
    dst = edge_index[1]
    x_p = jnp.pad(x, ((0, NP - N), (0, 0)))
    batch_p = jnp.pad(batch, (0, NP - N), constant_values=NUM_GRAPHS)

    deg0, deg1 = _deg_kernel(dst)
    dinv, y1 = _scale_call(deg0.reshape(NP, 1), deg1.reshape(NP, 1), x_p)
    z1p0, z1p1 = _agg1_kernel(y1, src, dst)
    y2a, y2b = _layer1_call(z1p0, z1p1, y1, dinv, W1, b1.reshape(1, HIDDEN))
    z2a, z2b = _agg2_kernel(y2a, y2b, src, dst)
    return _readout_call(z2a, z2b, y2a, y2b, dinv, batch_p.reshape(NP, 1),
                         W2, b2.reshape(1, HIDDEN), lin_W,
                         lin_b.reshape(1, NUM_CLASSES))


# speedup vs baseline: 1.4791x; 1.0076x over previous
import functools

import jax
import jax.numpy as jnp
from jax import lax
from jax.experimental import pallas as pl
from jax.experimental.pallas import tpu as pltpu
from jax.experimental.pallas import tpu_sc as plsc

N = 10000
NP = 10240
E = 320000
IN_DIM = 128
HIDDEN = 256
HH = HIDDEN // 2
NUM_CLASSES = 2
NUM_GRAPHS = 64

NC = 2
NS = 16
NW = NC * NS
EC = 80
ECA = 40
NBUF = 7
SLAB = NP // NS

_MESH = plsc.VectorSubcoreMesh(core_axis_name="c", subcore_axis_name="s")

ROW_BLK = 1024
GRID_R = NP // ROW_BLK


def _zero_vmem_1d(ref, n):
    def body(i, carry):
        ref[pl.ds(i * 16, 16)] = jnp.zeros((16,), jnp.float32)
        return carry
    lax.fori_loop(0, n // 16, body, 0)


@functools.partial(
    pl.kernel,
    out_type=(jax.ShapeDtypeStruct((NP,), jnp.float32),
              jax.ShapeDtypeStruct((NP,), jnp.float32)),
    mesh=_MESH,
    scratch_types=(
        pltpu.VMEM((E // NW,), jnp.int32),
        pltpu.VMEM((EC,), jnp.float32),
        pltpu.VMEM((SLAB,), jnp.float32),
        pltpu.VMEM_SHARED((NP,), jnp.float32),
        pltpu.SemaphoreType.DMA,
    ),
)
def _deg_kernel(dst_hbm, deg0_hbm, deg1_hbm, idx_v, ones_v, zeros_v, deg_sh, sem):
    c = lax.axis_index("c")
    s = lax.axis_index("s")
    for j in range(EC // 16):
        ones_v[pl.ds(j * 16, 16)] = jnp.ones((16,), jnp.float32)
    _zero_vmem_1d(zeros_v, SLAB)
    pltpu.sync_copy(zeros_v, deg_sh.at[pl.ds(s * SLAB, SLAB)])
    per_w = E // NW
    pltpu.sync_copy(dst_hbm.at[pl.ds((c * NS + s) * per_w, per_w)], idx_v)
    plsc.subcore_barrier()

    def fire(i, carry):
        pltpu.async_copy(ones_v, deg_sh.at[idx_v.at[pl.ds(i * EC, EC)]],
                         sem, add=True)
        return carry
    lax.fori_loop(0, per_w // EC, fire, 0)

    def drain(i, carry):
        pltpu.make_async_copy(ones_v,
                              deg_sh.at[idx_v.at[pl.ds(i * EC, EC)]],
                              sem).wait()
        return carry
    lax.fori_loop(0, per_w // EC, drain, 0)

    plsc.subcore_barrier()

    @pl.when(c == 0)
    def _():
        pltpu.sync_copy(deg_sh.at[pl.ds(s * SLAB, SLAB)],
                        deg0_hbm.at[pl.ds(s * SLAB, SLAB)])

    @pl.when(c == 1)
    def _():
        pltpu.sync_copy(deg_sh.at[pl.ds(s * SLAB, SLAB)],
                        deg1_hbm.at[pl.ds(s * SLAB, SLAB)])


def _agg_pipelined(y_hbm, src_hbm, dst_hbm, z_sh, idxs_v, idxd_v,
                   rows, semg, semc, ebase, n_e):
    pltpu.sync_copy(src_hbm.at[pl.ds(ebase, n_e)], idxs_v)
    pltpu.sync_copy(dst_hbm.at[pl.ds(ebase, n_e)], idxd_v)
    nch = n_e // ECA

    def s_idx(i):
        return idxs_v.at[pl.ds(i * ECA, ECA)]

    def d_idx(i):
        return idxd_v.at[pl.ds(i * ECA, ECA)]

    for b in range(NBUF):
        pltpu.async_copy(y_hbm.at[s_idx(b)], rows[b], semg[b])

    def step(i, b, bp):
        pltpu.make_async_copy(y_hbm.at[s_idx(i)], rows[b],
                              semg[b]).wait()
        pltpu.async_copy(rows[b], z_sh.at[d_idx(i)], semc[b],
                         add=True)
        ip = jnp.maximum(i - 1, 0)

        @pl.when(i >= 1)
        def _():
            pltpu.make_async_copy(rows[bp], z_sh.at[d_idx(ip)],
                                  semc[bp]).wait()

        @pl.when((i >= 1) & (i - 1 + NBUF < nch))
        def _():
            pltpu.async_copy(y_hbm.at[s_idx(ip + NBUF)], rows[bp],
                             semg[bp])

    def body(ib, carry):
        for b in range(NBUF):
            step(ib * NBUF + b, b, (b - 1) % NBUF)
        return carry
    lax.fori_loop(0, nch // NBUF, body, 0)
    for r in range(nch % NBUF):
        i = nch - (nch % NBUF) + r
        step(i, i % NBUF, (i - 1) % NBUF)
    pltpu.make_async_copy(rows[(nch - 1) % NBUF], z_sh.at[d_idx(nch - 1)],
                          semc[(nch - 1) % NBUF]).wait()


def _agg_prologue(z_sh, rows, s):
    def body(i, carry):
        rows[i // (IN_DIM // 16), pl.ds((i % (IN_DIM // 16)) * 16, 16)] = (
            jnp.zeros((16,), jnp.float32))
        return carry
    lax.fori_loop(0, ECA * IN_DIM // 16, body, 0)
    for j in range(SLAB // ECA):
        pltpu.sync_copy(rows, z_sh.at[pl.ds(s * SLAB + j * ECA, ECA)])
    plsc.subcore_barrier()


EPT = 5000


def _agg_scratch(n_e_per):
    return (
        pltpu.VMEM((n_e_per,), jnp.int32),
        pltpu.VMEM((n_e_per,), jnp.int32),
        *[pltpu.VMEM((ECA, IN_DIM), jnp.float32) for _ in range(NBUF)],
        pltpu.VMEM_SHARED((NP, IN_DIM), jnp.float32),
        *[pltpu.SemaphoreType.DMA for _ in range(2 * NBUF)],
    )


_AGG_OUT = (jax.ShapeDtypeStruct((NP, IN_DIM), jnp.float32),
            jax.ShapeDtypeStruct((NP, IN_DIM), jnp.float32))


@functools.partial(pl.kernel, out_type=_AGG_OUT, mesh=_MESH,
                   scratch_types=_agg_scratch(EPT))
def _agg1_kernel(y_hbm, src_hbm, dst_hbm, z0_hbm, z1_hbm,
                 idxs_v, idxd_v, *rs):
    rows, z_sh = rs[:NBUF], rs[NBUF]
    semg, semc = rs[NBUF + 1:NBUF + 1 + NBUF], rs[NBUF + 1 + NBUF:]
    c = lax.axis_index("c")
    s = lax.axis_index("s")
    _agg_prologue(z_sh, rows[0], s)
    ebase = (c * NS + s) * (E // NW)
    for p in range(E // NW // EPT):
        _agg_pipelined(y_hbm, src_hbm, dst_hbm, z_sh, idxs_v, idxd_v,
                       rows, semg, semc, ebase + p * EPT, EPT)
    plsc.subcore_barrier()

    @pl.when(c == 0)
    def _():
        pltpu.sync_copy(z_sh.at[pl.ds(s * SLAB, SLAB)],
                        z0_hbm.at[pl.ds(s * SLAB, SLAB)])

    @pl.when(c == 1)
    def _():
        pltpu.sync_copy(z_sh.at[pl.ds(s * SLAB, SLAB)],
                        z1_hbm.at[pl.ds(s * SLAB, SLAB)])


@functools.partial(pl.kernel, out_type=_AGG_OUT, mesh=_MESH,
                   scratch_types=_agg_scratch(EPT))
def _agg2_kernel(ya_hbm, yb_hbm, src_hbm, dst_hbm, za_hbm, zb_hbm,
                 idxs_v, idxd_v, *rs):
    rows, z_sh = rs[:NBUF], rs[NBUF]
    semg, semc = rs[NBUF + 1:NBUF + 1 + NBUF], rs[NBUF + 1 + NBUF:]
    c = lax.axis_index("c")
    s = lax.axis_index("s")
    _agg_prologue(z_sh, rows[0], s)
    ebase = s * (E // NS)

    @pl.when(c == 0)
    def _():
        for p in range(E // NS // EPT):
            _agg_pipelined(ya_hbm, src_hbm, dst_hbm, z_sh, idxs_v, idxd_v,
                           rows, semg, semc, ebase + p * EPT, EPT)

    @pl.when(c == 1)
    def _():
        for p in range(E // NS // EPT):
            _agg_pipelined(yb_hbm, src_hbm, dst_hbm, z_sh, idxs_v, idxd_v,
                           rows, semg, semc, ebase + p * EPT, EPT)

    plsc.subcore_barrier()

    @pl.when(c == 0)
    def _():
        pltpu.sync_copy(z_sh.at[pl.ds(s * SLAB, SLAB)],
                        za_hbm.at[pl.ds(s * SLAB, SLAB)])

    @pl.when(c == 1)
    def _():
        pltpu.sync_copy(z_sh.at[pl.ds(s * SLAB, SLAB)],
                        zb_hbm.at[pl.ds(s * SLAB, SLAB)])


def _scale_body(deg0_ref, deg1_ref, x_ref, dinv_ref, y1_ref):
    deg = deg0_ref[...] + deg1_ref[...] + 1.0
    dinv = lax.rsqrt(deg)
    dinv_ref[...] = dinv
    y1_ref[...] = x_ref[...] * dinv


_scale_call = pl.pallas_call(
    _scale_body,
    grid=(GRID_R,),
    in_specs=[
        pl.BlockSpec((ROW_BLK, 1), lambda i: (i, 0)),
        pl.BlockSpec((ROW_BLK, 1), lambda i: (i, 0)),
        pl.BlockSpec((ROW_BLK, IN_DIM), lambda i: (i, 0)),
    ],
    out_specs=[
        pl.BlockSpec((ROW_BLK, 1), lambda i: (i, 0)),
        pl.BlockSpec((ROW_BLK, IN_DIM), lambda i: (i, 0)),
    ],
    out_shape=[
        jax.ShapeDtypeStruct((NP, 1), jnp.float32),
        jax.ShapeDtypeStruct((NP, IN_DIM), jnp.float32),
    ],
)


def _layer1_body(z0_ref, z1_ref, y1_ref, dinv_ref, w1_ref, b1_ref,
                 ya_ref, yb_ref):
    dinv = dinv_ref[...]
    t = (z0_ref[...] + z1_ref[...] + y1_ref[...]) * dinv
    h = jnp.dot(t, w1_ref[...], preferred_element_type=jnp.float32)
    h = jnp.maximum(h + b1_ref[...], 0.0)
    y2 = h * dinv
    ya_ref[...] = y2[:, :HH]
    yb_ref[...] = y2[:, HH:]


_layer1_call = pl.pallas_call(
    _layer1_body,
    grid=(GRID_R,),
    in_specs=[
        pl.BlockSpec((ROW_BLK, IN_DIM), lambda i: (i, 0)),
        pl.BlockSpec((ROW_BLK, IN_DIM), lambda i: (i, 0)),
        pl.BlockSpec((ROW_BLK, IN_DIM), lambda i: (i, 0)),
        pl.BlockSpec((ROW_BLK, 1), lambda i: (i, 0)),
        pl.BlockSpec((IN_DIM, HIDDEN), lambda i: (0, 0)),
        pl.BlockSpec((1, HIDDEN), lambda i: (0, 0)),
    ],
    out_specs=[
        pl.BlockSpec((ROW_BLK, HH), lambda i: (i, 0)),
        pl.BlockSpec((ROW_BLK, HH), lambda i: (i, 0)),
    ],
    out_shape=[
        jax.ShapeDtypeStruct((NP, HH), jnp.float32),
        jax.ShapeDtypeStruct((NP, HH), jnp.float32),
    ],
)


def _readout_body(za_ref, zb_ref, ya_ref, yb_ref, dinv_ref, batch_ref,
                  w2_ref, b2_ref, lw_ref, lb_ref, out_ref, gsum, gmax):
    i = pl.program_id(0)

    @pl.when(i == 0)
    def _():
        gsum[...] = jnp.zeros_like(gsum)
        gmax[...] = jnp.zeros_like(gmax)

    dinv = dinv_ref[...]
    t = jnp.concatenate(
        [za_ref[...] + ya_ref[...], zb_ref[...] + yb_ref[...]], axis=1)
    t = t * dinv
    h = jnp.dot(t, w2_ref[...], preferred_element_type=jnp.float32)
    h = jnp.maximum(h + b2_ref[...], 0.0)

    b = batch_ref[...]
    onehot = (b == lax.broadcasted_iota(jnp.int32, (1, NUM_GRAPHS), 1)
              ).astype(jnp.float32)
    gsum[...] += lax.dot_general(onehot, h, (((0,), (0,)), ((), ())),
                                 preferred_element_type=jnp.float32)

    def seg_max(g, carry):
        v = jnp.max(jnp.where(b == g, h, 0.0), axis=0, keepdims=True)
        cur = gmax[pl.ds(g, 1), :]
        gmax[pl.ds(g, 1), :] = jnp.maximum(cur, v)
        return carry
    lo = batch_ref[0, 0]
    hi = jnp.minimum(batch_ref[ROW_BLK - 1, 0], NUM_GRAPHS - 1) + 1
    lax.fori_loop(lo, hi, seg_max, 0)

    @pl.when(i == GRID_R - 1)
    def _():
        g = jnp.concatenate([gsum[...], gmax[...]], axis=1)
        out_ref[...] = (jnp.dot(g, lw_ref[...],
                                preferred_element_type=jnp.float32)
                        + lb_ref[...])


_readout_call = pl.pallas_call(
    _readout_body,
    grid=(GRID_R,),
    in_specs=[
        pl.BlockSpec((ROW_BLK, HH), lambda i: (i, 0)),
        pl.BlockSpec((ROW_BLK, HH), lambda i: (i, 0)),
        pl.BlockSpec((ROW_BLK, HH), lambda i: (i, 0)),
        pl.BlockSpec((ROW_BLK, HH), lambda i: (i, 0)),
        pl.BlockSpec((ROW_BLK, 1), lambda i: (i, 0)),
        pl.BlockSpec((ROW_BLK, 1), lambda i: (i, 0)),
        pl.BlockSpec((HIDDEN, HIDDEN), lambda i: (0, 0)),
        pl.BlockSpec((1, HIDDEN), lambda i: (0, 0)),
        pl.BlockSpec((2 * HIDDEN, NUM_CLASSES), lambda i: (0, 0)),
        pl.BlockSpec((1, NUM_CLASSES), lambda i: (0, 0)),
    ],
    out_specs=pl.BlockSpec((NUM_GRAPHS, NUM_CLASSES), lambda i: (0, 0)),
    out_shape=jax.ShapeDtypeStruct((NUM_GRAPHS, NUM_CLASSES), jnp.float32),
    scratch_shapes=[
        pltpu.VMEM((NUM_GRAPHS, HIDDEN), jnp.float32),
        pltpu.VMEM((NUM_GRAPHS, HIDDEN), jnp.float32),
    ],
)


def kernel(x, edge_index, batch, W1, b1, W2, b2, lin_W, lin_b):
    src = edge_index[0]
    dst = edge_index[1]
    x_p = jnp.pad(x, ((0, NP - N), (0, 0)))
    batch_p = jnp.pad(batch, (0, NP - N), constant_values=NUM_GRAPHS)

    deg0, deg1 = _deg_kernel(dst)
    dinv, y1 = _scale_call(deg0.reshape(NP, 1), deg1.reshape(NP, 1), x_p)
    z1p0, z1p1 = _agg1_kernel(y1, src, dst)
    y2a, y2b = _layer1_call(z1p0, z1p1, y1, dinv, W1, b1.reshape(1, HIDDEN))
    z2a, z2b = _agg2_kernel(y2a, y2b, src, dst)
    return _readout_call(z2a, z2b, y2a, y2b, dinv, batch_p.reshape(NP, 1),
                         W2, b2.reshape(1, HIDDEN), lin_W,
                         lin_b.reshape(1, NUM_CLASSES))
